# baseline (speedup 1.0000x reference)
import functools

import jax
import jax.numpy as jnp
from jax import lax
from jax.experimental import pallas as pl
from jax.experimental.pallas import tpu as pltpu
from jax.experimental.pallas import tpu_sc as plsc

NUM_CORES = 2
NUM_SUBCORES = 16
NUM_WORKERS = NUM_CORES * NUM_SUBCORES
CHUNK_ROWS = 64


def kernel(tokens, embed_table):
    batch = tokens.shape[0]
    seq_len = tokens.shape[1]
    embed_dim = embed_table.shape[1]
    rows_per_worker = seq_len // NUM_WORKERS
    n_chunks = rows_per_worker // CHUNK_ROWS
    mesh = plsc.VectorSubcoreMesh(core_axis_name="c", subcore_axis_name="s")

    @functools.partial(
        pl.kernel,
        mesh=mesh,
        out_type=jax.ShapeDtypeStruct(
            (batch, seq_len, embed_dim), embed_table.dtype),
        scratch_types=[
            pltpu.VMEM((CHUNK_ROWS, embed_dim), jnp.float32),
        ],
    )
    def sc_copy(table_hbm, out_hbm, buf):
        wid = lax.axis_index("s") * NUM_CORES + lax.axis_index("c")
        base = wid * rows_per_worker

        def body(i, carry):
            r = base + i * CHUNK_ROWS
            pltpu.sync_copy(table_hbm.at[pl.ds(r, CHUNK_ROWS)], buf)
            for b in range(batch):
                pltpu.sync_copy(buf, out_hbm.at[b, pl.ds(r, CHUNK_ROWS)])
            return carry

        lax.fori_loop(0, n_chunks, body, 0)

    return sc_copy(embed_table[:seq_len])
